# baseline (speedup 1.0000x reference)
import functools

import jax
import jax.numpy as jnp
from jax import lax
from jax.experimental import pallas as pl
from jax.experimental.pallas import tpu as pltpu
from jax.experimental.pallas import tpu_sc as plsc

VOCABN = 1000000
EMB = 32
B = 4096
L = 200

NC = 2
NS = 16
NW = NC * NS
SPW = B // NW
TOK = SPW * L
NFULL = L // 16
TAIL = L - NFULL * 16
G1 = 128
G2 = L - G1

TCOLS = VOCABN // 128
VMAIN = TCOLS * 128
VTAIL = VOCABN - VMAIN
BASE_TC = TCOLS // NW
EXTRA_W = TCOLS - BASE_TC * NW
CW = 512
NCHUNK = BASE_TC * 128 // CW
CVALS = CW * EMB


def _conv_kernel(tt_hbm, tail_hbm, out_hbm, in0_v, in1_v, stage0_v,
                 stage1_v, xstage_v, tstage_v, tail_v, sem_i0, sem_i1,
                 sem_o0, sem_o1, sem_x, sem_t):
    c = lax.axis_index("c")
    s = lax.axis_index("s")
    wid = s * NC + c
    col_base = (wid * BASE_TC + jnp.minimum(wid, EXTRA_W)) * 128

    lane = lax.iota(jnp.int32, 16)
    lane32 = lane * EMB
    sem_i = (sem_i0, sem_i1)
    sem_o = (sem_o0, sem_o1)
    in_b = (in0_v, in1_v)
    stage_b = (stage0_v, stage1_v)

    def col0_of(k):
        return col_base + k * CW

    def issue_in(k, slot):
        pltpu.async_copy(tt_hbm.at[:, pl.ds(col0_of(k), CW)],
                         in_b[slot], sem_i[slot])

    def wait_in(slot):
        pltpu.make_async_copy(tt_hbm.at[:, pl.ds(0, CW)], in_b[slot],
                              sem_i[slot]).wait()

    def compute(k, slot):
        def group(g, carry):
            r032 = g * (16 * EMB)
            for c0 in range(EMB):
                vals = in_b[slot][c0, pl.ds(g * 16, 16)]
                plsc.store_scatter(stage_b[slot], [lane32 + (r032 + c0)],
                                   vals)
            return carry

        lax.fori_loop(0, CW // 16, group, 0)
        pltpu.async_copy(stage_b[slot],
                         out_hbm.at[pl.ds(col0_of(k) * EMB, CVALS)],
                         sem_o[slot])

    def wait_out(slot):
        pltpu.make_async_copy(stage_b[slot],
                              out_hbm.at[pl.ds(0, CVALS)],
                              sem_o[slot]).wait()

    issue_in(0, 0)
    issue_in(1, 1)

    def body(kp, carry):
        k0 = 2 * kp
        wait_in(0)

        @pl.when(kp > 0)
        def _():
            wait_out(0)

        compute(k0, 0)

        @pl.when(k0 + 2 < NCHUNK)
        def _():
            issue_in(k0 + 2, 0)

        wait_in(1)

        @pl.when(kp > 0)
        def _():
            wait_out(1)

        compute(k0 + 1, 1)

        @pl.when(k0 + 3 < NCHUNK)
        def _():
            issue_in(k0 + 3, 1)

        return carry

    lax.fori_loop(0, NCHUNK // 2, body, 0)

    wait_in(0)
    wait_out(0)
    compute(NCHUNK - 1, 0)

    @pl.when(wid < EXTRA_W)
    def _():
        xcol0 = col_base + BASE_TC * 128
        pltpu.sync_copy(tt_hbm.at[:, pl.ds(xcol0, 128)],
                        in0_v.at[:, pl.ds(0, 128)])

        def xgroup(g, carry):
            r032 = g * (16 * EMB)
            for c0 in range(EMB):
                vals = in0_v[c0, pl.ds(g * 16, 16)]
                plsc.store_scatter(xstage_v, [lane32 + (r032 + c0)], vals)
            return carry

        lax.fori_loop(0, 128 // 16, xgroup, 0)
        pltpu.async_copy(xstage_v, out_hbm.at[pl.ds(xcol0 * EMB, 128 * EMB)],
                         sem_x)
        pltpu.make_async_copy(xstage_v,
                              out_hbm.at[pl.ds(0, 128 * EMB)], sem_x).wait()

    @pl.when(wid == NW - 1)
    def _():
        pltpu.sync_copy(tail_hbm, tail_v)

        def trow(r, carry):
            tstage_v[pl.ds(r * EMB, 16)] = tail_v[r, pl.ds(0, 16)]
            tstage_v[pl.ds(r * EMB + 16, 16)] = tail_v[r, pl.ds(16, 16)]
            return carry

        lax.fori_loop(0, VTAIL, trow, 0)
        pltpu.async_copy(tstage_v,
                         out_hbm.at[pl.ds(VMAIN * EMB, VTAIL * EMB)], sem_t)
        pltpu.make_async_copy(tstage_v,
                              out_hbm.at[pl.ds(0, VTAIL * EMB)], sem_t).wait()

    wait_out(0)
    wait_out(1)


def _sc_kernel(x_hbm, table_hbm, out_hbm, idx_v, sid_v, cnt_v, rows_v, out_v,
               sem0, sem1):
    c = lax.axis_index("c")
    s = lax.axis_index("s")
    wid = s * NC + c
    base_tok = wid * TOK

    pltpu.sync_copy(x_hbm.at[pl.ds(base_tok, TOK)], idx_v.at[pl.ds(0, TOK)])

    lane = lax.iota(jnp.int32, 16)
    sems = (sem0, sem1)

    def issue(si, slot):
        sbase = si * L
        cnt = jnp.zeros((16,), jnp.int32)
        for k in range(NFULL + 1):
            v = idx_v[pl.ds(sbase + 16 * k, 16)]
            xm = jnp.where(v == 1, 0, v)
            if k == NFULL:
                valid = (xm != 0) & (lane < TAIL)
            else:
                valid = xm != 0
            cnt = cnt + plsc.all_reduce_population_count(valid)
            sid_v[slot, pl.ds(16 * k, 16)] = xm
        cnt_v[slot, pl.ds(0, 16)] = cnt
        pltpu.async_copy(table_hbm.at[sid_v.at[slot, pl.ds(0, G1)]],
                         rows_v.at[slot, pl.ds(0, G1)], sems[slot])
        pltpu.async_copy(table_hbm.at[sid_v.at[slot, pl.ds(G1, G2)]],
                         rows_v.at[slot, pl.ds(G1, G2)], sems[slot])

    def drain(si, slot):
        pltpu.make_async_copy(table_hbm.at[sid_v.at[slot, pl.ds(0, G1)]],
                              rows_v.at[slot, pl.ds(0, G1)],
                              sems[slot]).wait()
        pltpu.make_async_copy(table_hbm.at[sid_v.at[slot, pl.ds(G1, G2)]],
                              rows_v.at[slot, pl.ds(G1, G2)],
                              sems[slot]).wait()

        def red(o, accs):
            a0, a1 = accs
            for j in range(8):
                r = o * 8 + j
                a0 = a0 + rows_v[slot, r, pl.ds(0, 16)]
                a1 = a1 + rows_v[slot, r, pl.ds(16, 16)]
            return a0, a1

        acc0, acc1 = lax.fori_loop(
            0, L // 8, red,
            (jnp.zeros((16,), jnp.float32), jnp.zeros((16,), jnp.float32)))

        count = cnt_v[slot, pl.ds(0, 16)].astype(jnp.float32)
        scale = jnp.where(count > 0.0, 1.0 / jnp.maximum(count, 1.0), 0.0)
        out_v[si, pl.ds(0, 16)] = acc0 * scale
        out_v[si, pl.ds(16, 16)] = acc1 * scale

    issue(0, 0)

    def body(k, carry):
        s0 = 2 * k
        issue(s0 + 1, 1)
        drain(s0, 0)

        @pl.when(s0 + 2 < SPW)
        def _():
            issue(s0 + 2, 0)

        drain(s0 + 1, 1)
        return carry

    lax.fori_loop(0, SPW // 2, body, 0)

    pltpu.sync_copy(out_v, out_hbm.at[pl.ds(wid * SPW, SPW)])


@jax.jit
def _run(x_flat, table):
    mesh = plsc.VectorSubcoreMesh(core_axis_name="c", subcore_axis_name="s")

    tail_pad = jnp.pad(table[VMAIN:, :], ((0, 0), (0, 128 - EMB)))

    conv = functools.partial(
        pl.kernel,
        out_type=jax.ShapeDtypeStruct((VOCABN * EMB,), jnp.float32),
        mesh=mesh,
        compiler_params=pltpu.CompilerParams(needs_layout_passes=False,
                                             use_tc_tiling_on_sc=True),
        scratch_types=[
            pltpu.VMEM((EMB, CW), jnp.float32),
            pltpu.VMEM((EMB, CW), jnp.float32),
            pltpu.VMEM((CVALS,), jnp.float32),
            pltpu.VMEM((CVALS,), jnp.float32),
            pltpu.VMEM((128 * EMB,), jnp.float32),
            pltpu.VMEM((VTAIL * EMB,), jnp.float32),
            pltpu.VMEM((VTAIL, 128), jnp.float32),
            pltpu.SemaphoreType.DMA,
            pltpu.SemaphoreType.DMA,
            pltpu.SemaphoreType.DMA,
            pltpu.SemaphoreType.DMA,
            pltpu.SemaphoreType.DMA,
            pltpu.SemaphoreType.DMA,
        ],
    )(_conv_kernel)
    table_lin = conv(table.T, tail_pad).reshape(VOCABN, EMB)

    gather = functools.partial(
        pl.kernel,
        out_type=jax.ShapeDtypeStruct((B, EMB), jnp.float32),
        mesh=mesh,
        compiler_params=pltpu.CompilerParams(needs_layout_passes=False,
                                             use_tc_tiling_on_sc=False),
        scratch_types=[
            pltpu.VMEM((TOK + 16,), jnp.int32),
            pltpu.VMEM((2, 208), jnp.int32),
            pltpu.VMEM((2, 16), jnp.int32),
            pltpu.VMEM((2, L, EMB), jnp.float32),
            pltpu.VMEM((SPW, EMB), jnp.float32),
            pltpu.SemaphoreType.DMA,
            pltpu.SemaphoreType.DMA,
        ],
    )(_sc_kernel)
    return gather(x_flat, table_lin)


def kernel(x, table):
    return _run(x.reshape(-1), table)


# speedup vs baseline: 1.2468x; 1.2468x over previous
import functools

import jax
import jax.numpy as jnp
from jax import lax
from jax.experimental import pallas as pl
from jax.experimental.pallas import tpu as pltpu
from jax.experimental.pallas import tpu_sc as plsc

VOCABN = 1000000
EMB = 32
B = 4096
L = 200

NC = 2
NS = 16
NW = NC * NS
SPW = B // NW
TOK = SPW * L
NFULL = L // 16
TAIL = L - NFULL * 16
G1 = 128
G2 = L - G1

TCOLS = VOCABN // 128
VMAIN = TCOLS * 128
VTAIL = VOCABN - VMAIN
BASE_TC = TCOLS // NW
EXTRA_W = TCOLS - BASE_TC * NW
CW = 512
NCHUNK = BASE_TC * 128 // CW
CVALS = CW * EMB


def _conv_kernel(tt_hbm, tail_hbm, out_hbm, in0_v, in1_v, stage0_v,
                 stage1_v, xstage_v, tstage_v, tail_v, sem_i0, sem_i1,
                 sem_o0, sem_o1, sem_x, sem_t):
    c = lax.axis_index("c")
    s = lax.axis_index("s")
    wid = s * NC + c
    col_base = (wid * BASE_TC + jnp.minimum(wid, EXTRA_W)) * 128

    lane = lax.iota(jnp.int32, 16)
    lane32 = lane * EMB
    sem_i = (sem_i0, sem_i1)
    sem_o = (sem_o0, sem_o1)
    in_b = (in0_v, in1_v)
    stage_b = (stage0_v, stage1_v)

    def col0_of(k):
        return col_base + k * CW

    def issue_in(k, slot):
        pltpu.async_copy(tt_hbm.at[:, pl.ds(col0_of(k), CW)],
                         in_b[slot], sem_i[slot])

    def wait_in(slot):
        pltpu.make_async_copy(tt_hbm.at[:, pl.ds(0, CW)], in_b[slot],
                              sem_i[slot]).wait()

    def compute(k, slot):
        def group(g, carry):
            r032 = g * (16 * EMB)
            for c0 in range(0, EMB, 8):
                vals = [in_b[slot][c0 + d, pl.ds(g * 16, 16)]
                        for d in range(8)]
                for d in range(8):
                    plsc.store_scatter(stage_b[slot],
                                       [lane32 + (r032 + c0 + d)], vals[d])
            return carry

        lax.fori_loop(0, CW // 16, group, 0)
        pltpu.async_copy(stage_b[slot],
                         out_hbm.at[pl.ds(col0_of(k) * EMB, CVALS)],
                         sem_o[slot])

    def wait_out(slot):
        pltpu.make_async_copy(stage_b[slot],
                              out_hbm.at[pl.ds(0, CVALS)],
                              sem_o[slot]).wait()

    issue_in(0, 0)
    issue_in(1, 1)

    def body(kp, carry):
        k0 = 2 * kp
        wait_in(0)

        @pl.when(kp > 0)
        def _():
            wait_out(0)

        compute(k0, 0)

        @pl.when(k0 + 2 < NCHUNK)
        def _():
            issue_in(k0 + 2, 0)

        wait_in(1)

        @pl.when(kp > 0)
        def _():
            wait_out(1)

        compute(k0 + 1, 1)

        @pl.when(k0 + 3 < NCHUNK)
        def _():
            issue_in(k0 + 3, 1)

        return carry

    lax.fori_loop(0, NCHUNK // 2, body, 0)

    wait_in(0)
    wait_out(0)
    compute(NCHUNK - 1, 0)

    @pl.when(wid < EXTRA_W)
    def _():
        xcol0 = col_base + BASE_TC * 128
        pltpu.sync_copy(tt_hbm.at[:, pl.ds(xcol0, 128)],
                        in0_v.at[:, pl.ds(0, 128)])

        def xgroup(g, carry):
            r032 = g * (16 * EMB)
            for c0 in range(0, EMB, 8):
                vals = [in0_v[c0 + d, pl.ds(g * 16, 16)] for d in range(8)]
                for d in range(8):
                    plsc.store_scatter(xstage_v,
                                       [lane32 + (r032 + c0 + d)], vals[d])
            return carry

        lax.fori_loop(0, 128 // 16, xgroup, 0)
        pltpu.async_copy(xstage_v, out_hbm.at[pl.ds(xcol0 * EMB, 128 * EMB)],
                         sem_x)
        pltpu.make_async_copy(xstage_v,
                              out_hbm.at[pl.ds(0, 128 * EMB)], sem_x).wait()

    @pl.when(wid == NW - 1)
    def _():
        pltpu.sync_copy(tail_hbm, tail_v)

        def trow(r, carry):
            tstage_v[pl.ds(r * EMB, 16)] = tail_v[r, pl.ds(0, 16)]
            tstage_v[pl.ds(r * EMB + 16, 16)] = tail_v[r, pl.ds(16, 16)]
            return carry

        lax.fori_loop(0, VTAIL, trow, 0)
        pltpu.async_copy(tstage_v,
                         out_hbm.at[pl.ds(VMAIN * EMB, VTAIL * EMB)], sem_t)
        pltpu.make_async_copy(tstage_v,
                              out_hbm.at[pl.ds(0, VTAIL * EMB)], sem_t).wait()

    wait_out(0)
    wait_out(1)


def _sc_kernel(x_hbm, table_hbm, out_hbm, idx_v, sid_v, cnt_v, rows_v, out_v,
               sem0, sem1):
    c = lax.axis_index("c")
    s = lax.axis_index("s")
    wid = s * NC + c
    base_tok = wid * TOK

    pltpu.sync_copy(x_hbm.at[pl.ds(base_tok, TOK)], idx_v.at[pl.ds(0, TOK)])

    lane = lax.iota(jnp.int32, 16)
    sems = (sem0, sem1)

    def issue(si, slot):
        sbase = si * L
        cnt = jnp.zeros((16,), jnp.int32)
        for k in range(NFULL + 1):
            v = idx_v[pl.ds(sbase + 16 * k, 16)]
            xm = jnp.where(v == 1, 0, v)
            if k == NFULL:
                valid = (xm != 0) & (lane < TAIL)
            else:
                valid = xm != 0
            cnt = cnt + plsc.all_reduce_population_count(valid)
            sid_v[slot, pl.ds(16 * k, 16)] = xm
        cnt_v[slot, pl.ds(0, 16)] = cnt
        pltpu.async_copy(table_hbm.at[sid_v.at[slot, pl.ds(0, G1)]],
                         rows_v.at[slot, pl.ds(0, G1)], sems[slot])
        pltpu.async_copy(table_hbm.at[sid_v.at[slot, pl.ds(G1, G2)]],
                         rows_v.at[slot, pl.ds(G1, G2)], sems[slot])

    def drain(si, slot):
        pltpu.make_async_copy(table_hbm.at[sid_v.at[slot, pl.ds(0, G1)]],
                              rows_v.at[slot, pl.ds(0, G1)],
                              sems[slot]).wait()
        pltpu.make_async_copy(table_hbm.at[sid_v.at[slot, pl.ds(G1, G2)]],
                              rows_v.at[slot, pl.ds(G1, G2)],
                              sems[slot]).wait()

        def red(o, accs):
            a0, a1 = accs
            for j in range(8):
                r = o * 8 + j
                a0 = a0 + rows_v[slot, r, pl.ds(0, 16)]
                a1 = a1 + rows_v[slot, r, pl.ds(16, 16)]
            return a0, a1

        acc0, acc1 = lax.fori_loop(
            0, L // 8, red,
            (jnp.zeros((16,), jnp.float32), jnp.zeros((16,), jnp.float32)))

        count = cnt_v[slot, pl.ds(0, 16)].astype(jnp.float32)
        scale = jnp.where(count > 0.0, 1.0 / jnp.maximum(count, 1.0), 0.0)
        out_v[si, pl.ds(0, 16)] = acc0 * scale
        out_v[si, pl.ds(16, 16)] = acc1 * scale

    issue(0, 0)

    def body(k, carry):
        s0 = 2 * k
        issue(s0 + 1, 1)
        drain(s0, 0)

        @pl.when(s0 + 2 < SPW)
        def _():
            issue(s0 + 2, 0)

        drain(s0 + 1, 1)
        return carry

    lax.fori_loop(0, SPW // 2, body, 0)

    pltpu.sync_copy(out_v, out_hbm.at[pl.ds(wid * SPW, SPW)])


@jax.jit
def _run(x_flat, table):
    mesh = plsc.VectorSubcoreMesh(core_axis_name="c", subcore_axis_name="s")

    tail_pad = jnp.pad(table[VMAIN:, :], ((0, 0), (0, 128 - EMB)))

    conv = functools.partial(
        pl.kernel,
        out_type=jax.ShapeDtypeStruct((VOCABN * EMB,), jnp.float32),
        mesh=mesh,
        compiler_params=pltpu.CompilerParams(needs_layout_passes=False,
                                             use_tc_tiling_on_sc=True),
        scratch_types=[
            pltpu.VMEM((EMB, CW), jnp.float32),
            pltpu.VMEM((EMB, CW), jnp.float32),
            pltpu.VMEM((CVALS,), jnp.float32),
            pltpu.VMEM((CVALS,), jnp.float32),
            pltpu.VMEM((128 * EMB,), jnp.float32),
            pltpu.VMEM((VTAIL * EMB,), jnp.float32),
            pltpu.VMEM((VTAIL, 128), jnp.float32),
            pltpu.SemaphoreType.DMA,
            pltpu.SemaphoreType.DMA,
            pltpu.SemaphoreType.DMA,
            pltpu.SemaphoreType.DMA,
            pltpu.SemaphoreType.DMA,
            pltpu.SemaphoreType.DMA,
        ],
    )(_conv_kernel)
    table_lin = conv(table.T, tail_pad).reshape(VOCABN, EMB)

    gather = functools.partial(
        pl.kernel,
        out_type=jax.ShapeDtypeStruct((B, EMB), jnp.float32),
        mesh=mesh,
        compiler_params=pltpu.CompilerParams(needs_layout_passes=False,
                                             use_tc_tiling_on_sc=False),
        scratch_types=[
            pltpu.VMEM((TOK + 16,), jnp.int32),
            pltpu.VMEM((2, 208), jnp.int32),
            pltpu.VMEM((2, 16), jnp.int32),
            pltpu.VMEM((2, L, EMB), jnp.float32),
            pltpu.VMEM((SPW, EMB), jnp.float32),
            pltpu.SemaphoreType.DMA,
            pltpu.SemaphoreType.DMA,
        ],
    )(_sc_kernel)
    return gather(x_flat, table_lin)


def kernel(x, table):
    return _run(x.reshape(-1), table)


# speedup vs baseline: 2.7939x; 2.2408x over previous
import functools

import jax
import jax.numpy as jnp
from jax import lax
from jax.experimental import pallas as pl
from jax.experimental.pallas import tpu as pltpu
from jax.experimental.pallas import tpu_sc as plsc

VOCABN = 1000000
EMB = 32
B = 4096
L = 200

NC = 2
NS = 16
NW = NC * NS
SPW = B // NW
TOK = SPW * L
NFULL = L // 16
TAIL = L - NFULL * 16
G1 = 128
G2 = L - G1

TCOLS = VOCABN // 128
VMAIN = TCOLS * 128
VTAIL = VOCABN - VMAIN
BASE_TC = TCOLS // NW
EXTRA_W = TCOLS - BASE_TC * NW
CW = 512
NCHUNK = BASE_TC * 128 // CW
CVALS = CW * EMB
SKEW = CW + 1


def _conv_kernel(tt_hbm, tail_hbm, out_hbm, in0_v, in1_v, stage0_v,
                 stage1_v, col_v, xstage_v, tstage_v, tail_v, sem_i0,
                 sem_i1, sem_o0, sem_o1, sem_x, sem_t):
    c = lax.axis_index("c")
    s = lax.axis_index("s")
    wid = s * NC + c
    col_base = (wid * BASE_TC + jnp.minimum(wid, EXTRA_W)) * 128

    lane = lax.iota(jnp.int32, 16)
    lane_skew = lane * SKEW
    sem_i = (sem_i0, sem_i1)
    sem_o = (sem_o0, sem_o1)
    in_b = (in0_v, in1_v)
    stage_b = (stage0_v, stage1_v)

    def col0_of(k):
        return col_base + k * CW

    def issue_in(k, slot):
        pltpu.async_copy(tt_hbm.at[:, pl.ds(col0_of(k), CW)],
                         in_b[slot], sem_i[slot])

    def wait_in(slot):
        pltpu.make_async_copy(tt_hbm.at[:, pl.ds(0, CW)], in_b[slot],
                              sem_i[slot]).wait()

    def compute(k, slot):
        def derow(c0, carry):
            base = c0 * SKEW
            for g0 in range(0, CW // 16, 8):
                vals = [in_b[slot][c0, pl.ds((g0 + d) * 16, 16)]
                        for d in range(8)]
                for d in range(8):
                    col_v[pl.ds(base + (g0 + d) * 16, 16)] = vals[d]
            return carry

        lax.fori_loop(0, EMB, derow, 0)

        def pass2(q, carry):
            t0 = q * 4
            gs = []
            for dt in range(4):
                idx0 = lane_skew + (t0 + dt)
                gs.append(plsc.load_gather(col_v, [idx0]))
                gs.append(plsc.load_gather(col_v, [idx0 + 16 * SKEW]))
            for dt in range(4):
                stage_b[slot][pl.ds((t0 + dt) * EMB, 16)] = gs[2 * dt]
                stage_b[slot][pl.ds((t0 + dt) * EMB + 16, 16)] = gs[2 * dt + 1]
            return carry

        lax.fori_loop(0, CW // 4, pass2, 0)
        pltpu.async_copy(stage_b[slot],
                         out_hbm.at[pl.ds(col0_of(k) * EMB, CVALS)],
                         sem_o[slot])

    def wait_out(slot):
        pltpu.make_async_copy(stage_b[slot],
                              out_hbm.at[pl.ds(0, CVALS)],
                              sem_o[slot]).wait()

    issue_in(0, 0)
    issue_in(1, 1)

    def body(kp, carry):
        k0 = 2 * kp
        wait_in(0)

        @pl.when(kp > 0)
        def _():
            wait_out(0)

        compute(k0, 0)

        @pl.when(k0 + 2 < NCHUNK)
        def _():
            issue_in(k0 + 2, 0)

        wait_in(1)

        @pl.when(kp > 0)
        def _():
            wait_out(1)

        compute(k0 + 1, 1)

        @pl.when(k0 + 3 < NCHUNK)
        def _():
            issue_in(k0 + 3, 1)

        return carry

    lax.fori_loop(0, NCHUNK // 2, body, 0)

    wait_in(0)
    wait_out(0)
    compute(NCHUNK - 1, 0)

    @pl.when(wid < EXTRA_W)
    def _():
        xcol0 = col_base + BASE_TC * 128
        pltpu.sync_copy(tt_hbm.at[:, pl.ds(xcol0, 128)],
                        in0_v.at[:, pl.ds(0, 128)])

        def xderow(c0, carry):
            base = c0 * SKEW
            vals = [in0_v[c0, pl.ds(d * 16, 16)] for d in range(8)]
            for d in range(8):
                col_v[pl.ds(base + d * 16, 16)] = vals[d]
            return carry

        lax.fori_loop(0, EMB, xderow, 0)

        def xpass2(q, carry):
            t0 = q * 4
            gs = []
            for dt in range(4):
                idx0 = lane_skew + (t0 + dt)
                gs.append(plsc.load_gather(col_v, [idx0]))
                gs.append(plsc.load_gather(col_v, [idx0 + 16 * SKEW]))
            for dt in range(4):
                xstage_v[pl.ds((t0 + dt) * EMB, 16)] = gs[2 * dt]
                xstage_v[pl.ds((t0 + dt) * EMB + 16, 16)] = gs[2 * dt + 1]
            return carry

        lax.fori_loop(0, 128 // 4, xpass2, 0)
        pltpu.async_copy(xstage_v, out_hbm.at[pl.ds(xcol0 * EMB, 128 * EMB)],
                         sem_x)
        pltpu.make_async_copy(xstage_v,
                              out_hbm.at[pl.ds(0, 128 * EMB)], sem_x).wait()

    @pl.when(wid == NW - 1)
    def _():
        pltpu.sync_copy(tail_hbm, tail_v)

        def trow(r, carry):
            tstage_v[pl.ds(r * EMB, 16)] = tail_v[r, pl.ds(0, 16)]
            tstage_v[pl.ds(r * EMB + 16, 16)] = tail_v[r, pl.ds(16, 16)]
            return carry

        lax.fori_loop(0, VTAIL, trow, 0)
        pltpu.async_copy(tstage_v,
                         out_hbm.at[pl.ds(VMAIN * EMB, VTAIL * EMB)], sem_t)
        pltpu.make_async_copy(tstage_v,
                              out_hbm.at[pl.ds(0, VTAIL * EMB)], sem_t).wait()

    wait_out(0)
    wait_out(1)


def _sc_kernel(x_hbm, table_hbm, out_hbm, idx_v, sid_v, cnt_v, rows_v, out_v,
               sem0, sem1):
    c = lax.axis_index("c")
    s = lax.axis_index("s")
    wid = s * NC + c
    base_tok = wid * TOK

    pltpu.sync_copy(x_hbm.at[pl.ds(base_tok, TOK)], idx_v.at[pl.ds(0, TOK)])

    lane = lax.iota(jnp.int32, 16)
    sems = (sem0, sem1)

    def issue(si, slot):
        sbase = si * L
        cnt = jnp.zeros((16,), jnp.int32)
        for k in range(NFULL + 1):
            v = idx_v[pl.ds(sbase + 16 * k, 16)]
            xm = jnp.where(v == 1, 0, v)
            if k == NFULL:
                valid = (xm != 0) & (lane < TAIL)
            else:
                valid = xm != 0
            cnt = cnt + plsc.all_reduce_population_count(valid)
            sid_v[slot, pl.ds(16 * k, 16)] = xm
        cnt_v[slot, pl.ds(0, 16)] = cnt
        pltpu.async_copy(table_hbm.at[sid_v.at[slot, pl.ds(0, G1)]],
                         rows_v.at[slot, pl.ds(0, G1)], sems[slot])
        pltpu.async_copy(table_hbm.at[sid_v.at[slot, pl.ds(G1, G2)]],
                         rows_v.at[slot, pl.ds(G1, G2)], sems[slot])

    def drain(si, slot):
        pltpu.make_async_copy(table_hbm.at[sid_v.at[slot, pl.ds(0, G1)]],
                              rows_v.at[slot, pl.ds(0, G1)],
                              sems[slot]).wait()
        pltpu.make_async_copy(table_hbm.at[sid_v.at[slot, pl.ds(G1, G2)]],
                              rows_v.at[slot, pl.ds(G1, G2)],
                              sems[slot]).wait()

        def red(o, accs):
            a0, a1 = accs
            for j in range(8):
                r = o * 8 + j
                a0 = a0 + rows_v[slot, r, pl.ds(0, 16)]
                a1 = a1 + rows_v[slot, r, pl.ds(16, 16)]
            return a0, a1

        acc0, acc1 = lax.fori_loop(
            0, L // 8, red,
            (jnp.zeros((16,), jnp.float32), jnp.zeros((16,), jnp.float32)))

        count = cnt_v[slot, pl.ds(0, 16)].astype(jnp.float32)
        scale = jnp.where(count > 0.0, 1.0 / jnp.maximum(count, 1.0), 0.0)
        out_v[si, pl.ds(0, 16)] = acc0 * scale
        out_v[si, pl.ds(16, 16)] = acc1 * scale

    issue(0, 0)

    def body(k, carry):
        s0 = 2 * k
        issue(s0 + 1, 1)
        drain(s0, 0)

        @pl.when(s0 + 2 < SPW)
        def _():
            issue(s0 + 2, 0)

        drain(s0 + 1, 1)
        return carry

    lax.fori_loop(0, SPW // 2, body, 0)

    pltpu.sync_copy(out_v, out_hbm.at[pl.ds(wid * SPW, SPW)])


@jax.jit
def _run(x_flat, table):
    mesh = plsc.VectorSubcoreMesh(core_axis_name="c", subcore_axis_name="s")

    tail_pad = jnp.pad(table[VMAIN:, :], ((0, 0), (0, 128 - EMB)))

    conv = functools.partial(
        pl.kernel,
        out_type=jax.ShapeDtypeStruct((VOCABN * EMB,), jnp.float32),
        mesh=mesh,
        compiler_params=pltpu.CompilerParams(needs_layout_passes=False,
                                             use_tc_tiling_on_sc=True),
        scratch_types=[
            pltpu.VMEM((EMB, CW), jnp.float32),
            pltpu.VMEM((EMB, CW), jnp.float32),
            pltpu.VMEM((CVALS,), jnp.float32),
            pltpu.VMEM((CVALS,), jnp.float32),
            pltpu.VMEM((EMB * SKEW,), jnp.float32),
            pltpu.VMEM((128 * EMB,), jnp.float32),
            pltpu.VMEM((VTAIL * EMB,), jnp.float32),
            pltpu.VMEM((VTAIL, 128), jnp.float32),
            pltpu.SemaphoreType.DMA,
            pltpu.SemaphoreType.DMA,
            pltpu.SemaphoreType.DMA,
            pltpu.SemaphoreType.DMA,
            pltpu.SemaphoreType.DMA,
            pltpu.SemaphoreType.DMA,
        ],
    )(_conv_kernel)
    table_lin = conv(table.T, tail_pad).reshape(VOCABN, EMB)

    gather = functools.partial(
        pl.kernel,
        out_type=jax.ShapeDtypeStruct((B, EMB), jnp.float32),
        mesh=mesh,
        compiler_params=pltpu.CompilerParams(needs_layout_passes=False,
                                             use_tc_tiling_on_sc=False),
        scratch_types=[
            pltpu.VMEM((TOK + 16,), jnp.int32),
            pltpu.VMEM((2, 208), jnp.int32),
            pltpu.VMEM((2, 16), jnp.int32),
            pltpu.VMEM((2, L, EMB), jnp.float32),
            pltpu.VMEM((SPW, EMB), jnp.float32),
            pltpu.SemaphoreType.DMA,
            pltpu.SemaphoreType.DMA,
        ],
    )(_sc_kernel)
    return gather(x_flat, table_lin)


def kernel(x, table):
    return _run(x.reshape(-1), table)
